# baseline (speedup 1.0000x reference)
import functools
import jax
import jax.numpy as jnp
import numpy as np
from jax import lax
from jax.experimental import pallas as pl
from jax.experimental.pallas import tpu as pltpu
from jax.experimental.pallas import tpu_sc as plsc

N = 10000
E = 320000
DIN = 128
HID = 256
HEADS = 8
NCLS = 40

NPAD = 10240
EPAD = 330240
EXTRA = 16
NC, NS = 2, 16
NW = NC * NS
EW1 = E // NW
EW2 = EPAD // NW
CH1 = 200
NCH1 = EW1 // CH1
C4 = 3840
NC4 = EPAD // C4
G4 = 48
RNG = NPAD // NW
ZR = 80
C3 = 4000
NC3 = E // C3
G3 = 128
TB = 2048
NTB = NPAD // TB
EB = 2000
NEB = E // EB

_mesh = plsc.VectorSubcoreMesh(
    core_axis_name="c", subcore_axis_name="s", num_cores=NC, num_subcores=NS)

f32 = jnp.float32
i32 = jnp.int32


def _wid():
    return lax.axis_index("s") * NC + lax.axis_index("c")


def _tc_prep_body(x_ref, wr_ref, wq_ref, c1_ref, t_ref):
    x = x_ref[...]
    r = jnp.dot(x, wr_ref[...], preferred_element_type=f32) + c1_ref[...]
    q = jnp.dot(x, wq_ref[...], preferred_element_type=f32)
    t_ref[...] = jnp.concatenate([r, q], axis=1)


def _tc_prep(xp, wr, wq, c1):
    return pl.pallas_call(
        _tc_prep_body,
        grid=(NTB,),
        in_specs=[
            pl.BlockSpec((TB, DIN), lambda i: (i, 0)),
            pl.BlockSpec((DIN, 64), lambda i: (0, 0)),
            pl.BlockSpec((DIN, 64), lambda i: (0, 0)),
            pl.BlockSpec((1, 64), lambda i: (0, 0)),
        ],
        out_specs=pl.BlockSpec((TB, 128), lambda i: (i, 0)),
        out_shape=jax.ShapeDtypeStruct((NPAD, 128), f32),
    )(xp, wr, wq, c1)


@functools.partial(
    pl.kernel,
    out_type=jax.ShapeDtypeStruct((E, 64), f32),
    mesh=_mesh,
    compiler_params=pltpu.CompilerParams(needs_layout_passes=False),
    scratch_types=[
        pltpu.VMEM((CH1,), i32),
        pltpu.VMEM((CH1,), i32),
        pltpu.VMEM((CH1, 128), f32),
        pltpu.VMEM((CH1, 128), f32),
        pltpu.VMEM((CH1, 64), f32),
        pltpu.SemaphoreType.DMA,
        pltpu.SemaphoreType.DMA,
    ],
)
def _sc_edge(t, src, dst, h1, sidx, didx, db, sb, ob, sem1, sem2):
    base = _wid() * EW1

    def chunk(ci, _):
        b = base + ci * CH1
        pltpu.sync_copy(src.at[pl.ds(b, CH1)], sidx)
        pltpu.sync_copy(dst.at[pl.ds(b, CH1)], didx)
        cp1 = pltpu.async_copy(t.at[didx], db, sem1)
        cp2 = pltpu.async_copy(t.at[sidx], sb, sem2)
        cp1.wait()
        cp2.wait()

        def edge(e, _):
            for k in range(4):
                v = db[e, pl.ds(16 * k, 16)] + sb[e, pl.ds(64 + 16 * k, 16)]
                ob[e, pl.ds(16 * k, 16)] = jnp.maximum(v, 0.0)
            return 0

        lax.fori_loop(0, CH1, edge, 0)
        pltpu.sync_copy(ob, h1.at[pl.ds(b, CH1)])
        return 0

    lax.fori_loop(0, NCH1, chunk, 0)


def _tc_mlp_body(h1_ref, w2_ref, c2_ref, w3_ref, c3_ref, o_ref):
    h2 = jnp.maximum(
        jnp.dot(h1_ref[...], w2_ref[...], preferred_element_type=f32)
        + c2_ref[...], 0.0)
    o_ref[...] = jnp.maximum(
        jnp.dot(h2, w3_ref[...], preferred_element_type=f32) + c3_ref[...], 0.0)


def _tc_mlp(h1, w2, c2, w3, c3):
    return pl.pallas_call(
        _tc_mlp_body,
        grid=(NEB,),
        in_specs=[
            pl.BlockSpec((EB, 64), lambda i: (i, 0)),
            pl.BlockSpec((64, 64), lambda i: (0, 0)),
            pl.BlockSpec((1, 64), lambda i: (0, 0)),
            pl.BlockSpec((64, 64), lambda i: (0, 0)),
            pl.BlockSpec((1, 64), lambda i: (0, 0)),
        ],
        out_specs=pl.BlockSpec((EB, 64), lambda i: (i, 0)),
        out_shape=jax.ShapeDtypeStruct((E, 64), f32),
    )(h1, w2, c2, w3, c3)


@functools.partial(
    pl.kernel,
    out_type=jax.ShapeDtypeStruct((NPAD, 64), f32),
    mesh=_mesh,
    compiler_params=pltpu.CompilerParams(needs_layout_passes=False),
    scratch_types=[
        pltpu.VMEM((C3,), i32),
        pltpu.VMEM((C3 + 144,), i32),
        pltpu.VMEM((G3,), i32),
        pltpu.VMEM((G3,), i32),
        pltpu.VMEM((G3, 128), f32),
        pltpu.VMEM((G3, 128), f32),
        pltpu.VMEM((RNG + 8, 64), f32),
        pltpu.SemaphoreType.DMA,
        pltpu.SemaphoreType.DMA,
    ],
)
def _sc_segmax(dst, h3p, x0, dbuf, pbuf, gidxa, gidxb, rowsa, rowsb, acc,
               sema, semb):
    lo = _wid() * RNG
    hi = lo + RNG
    zero = jnp.zeros((16,), f32)
    zeroi = jnp.zeros((16,), i32)

    def zr(r, _):
        for k in range(4):
            acc[r, pl.ds(16 * k, 16)] = zero
        return 0

    lax.fori_loop(0, RNG + 8, zr, 0)

    def zp(j, _):
        pbuf[pl.ds(j * 16, 16)] = zeroi
        return 0

    lax.fori_loop(0, (C3 + 144) // 16, zp, 0)

    def chunk(ci, _):
        b = ci * C3
        pltpu.sync_copy(dst.at[pl.ds(b, C3)], dbuf)

        def cvec(j, cnt):
            d = dbuf[pl.ds(j * 16, 16)]
            m = (d >= lo) & (d < hi)
            eid = lax.iota(i32, 16) + (b + j * 16)
            packed = ((d - lo) << 20) | eid
            pos = plsc.cumsum(m.astype(i32))
            plsc.store_scatter(pbuf, [cnt + pos - 1], packed, mask=m)
            return cnt + pos[15]

        m_total = lax.fori_loop(0, C3 // 16, cvec, 0)
        sent = lax.iota(i32, 16) * 0 + (RNG << 20)

        def st(t_, _):
            pbuf[pl.ds(m_total + t_ * 16, 16)] = sent
            return 0

        lax.fori_loop(0, 8, st, 0)

        def build_issue(k, gidx, rows, sem):
            def ext(t_, _2):
                p = pbuf[pl.ds(k * G3 + t_ * 16, 16)]
                gidx[pl.ds(t_ * 16, 16)] = (p & 0xFFFFF) >> 1
                return 0

            lax.fori_loop(0, G3 // 16, ext, 0)
            pltpu.async_copy(h3p.at[gidx], rows, sem)

        def process(k, rows):
            def grp(g, _2):
                base = k * G3 + g * 16
                pv = pbuf[pl.ds(base, 16)]
                for jj in range(16):
                    p = pv[jj]
                    dl = p >> 20
                    parb = (p & 1) > 0
                    j = g * 16 + jj
                    for q in range(4):
                        v = jnp.where(parb,
                                      rows[j, pl.ds(64 + 16 * q, 16)],
                                      rows[j, pl.ds(16 * q, 16)])
                        acc[dl, pl.ds(16 * q, 16)] = jnp.maximum(
                            acc[dl, pl.ds(16 * q, 16)], v)
                return 0

            lax.fori_loop(0, G3 // 16, grp, 0)

        @pl.when(0 < m_total)
        def _():
            build_issue(0, gidxa, rowsa, sema)

        def pair(kk, _):
            k0 = kk * 2
            k1 = k0 + 1

            @pl.when(k0 * G3 < m_total)
            def _():
                pltpu.make_async_copy(h3p.at[gidxa], rowsa, sema).wait()

                @pl.when(k1 * G3 < m_total)
                def _2():
                    build_issue(k1, gidxb, rowsb, semb)

                process(k0, rowsa)

                @pl.when(k1 * G3 < m_total)
                def _3():
                    pltpu.make_async_copy(h3p.at[gidxb], rowsb, semb).wait()

                    @pl.when((k0 + 2) * G3 < m_total)
                    def _4():
                        build_issue(k0 + 2, gidxa, rowsa, sema)

                    process(k1, rowsb)
            return 0

        lax.fori_loop(0, (C3 // G3 + 2) // 2, pair, 0)
        return 0

    lax.fori_loop(0, NC3, chunk, 0)
    pltpu.sync_copy(acc.at[pl.ds(0, RNG)], x0.at[pl.ds(lo, RNG)])


def _gat_pre(i, xr, w_ref, atts_ref, attd_ref, hm_ref, wl_ref, bl_ref,
             h_ref, ass_ref, asd_ref, mxs_ref, mxd_ref, skn_ref,
             valid):
    h = jnp.dot(xr, w_ref[...], preferred_element_type=f32)
    h_ref[...] = h
    asv = jnp.dot(h * atts_ref[...], hm_ref[...], preferred_element_type=f32)
    adv = jnp.dot(h * attd_ref[...], hm_ref[...], preferred_element_type=f32)
    asv = jnp.where(valid, asv, -1e30)
    adv = jnp.where(valid, adv, -1e30)
    ass_ref[...] = asv
    asd_ref[...] = adv

    @pl.when(i == 0)
    def _():
        mxs_ref[...] = jnp.full((1, 16), -1e30, f32)
        mxd_ref[...] = jnp.full((1, 16), -1e30, f32)

    mxs_ref[...] = jnp.maximum(mxs_ref[...], jnp.max(asv, axis=0, keepdims=True))
    mxd_ref[...] = jnp.maximum(mxd_ref[...], jnp.max(adv, axis=0, keepdims=True))
    skn_ref[...] = jnp.dot(xr, wl_ref[...], preferred_element_type=f32) + bl_ref[...]


def _pre_specs(din):
    ins = [
        pl.BlockSpec((din, HID), lambda i: (0, 0)),
        pl.BlockSpec((1, HID), lambda i: (0, 0)),
        pl.BlockSpec((1, HID), lambda i: (0, 0)),
        pl.BlockSpec((HID, 16), lambda i: (0, 0)),
        pl.BlockSpec((din, HID), lambda i: (0, 0)),
        pl.BlockSpec((1, HID), lambda i: (0, 0)),
    ]
    outs = [
        pl.BlockSpec((TB, HID), lambda i: (i, 0)),
        pl.BlockSpec((TB, 16), lambda i: (i, 0)),
        pl.BlockSpec((TB, 16), lambda i: (i, 0)),
        pl.BlockSpec((1, 16), lambda i: (0, 0)),
        pl.BlockSpec((1, 16), lambda i: (0, 0)),
        pl.BlockSpec((TB, HID), lambda i: (i, 0)),
    ]
    oshapes = [
        jax.ShapeDtypeStruct((NPAD, HID), f32),
        jax.ShapeDtypeStruct((NPAD, 16), f32),
        jax.ShapeDtypeStruct((NPAD, 16), f32),
        jax.ShapeDtypeStruct((1, 16), f32),
        jax.ShapeDtypeStruct((1, 16), f32),
        jax.ShapeDtypeStruct((NPAD, HID), f32),
    ]
    return ins, outs, oshapes


def _tc_gat_first_body(x0_ref, w_ref, atts_ref, attd_ref, hm_ref, wl_ref,
                       bl_ref, h_ref, ass_ref, asd_ref, mxs_ref,
                       mxd_ref, skn_ref):
    i = pl.program_id(0)
    rows = i * TB + lax.broadcasted_iota(i32, (TB, 1), 0)
    valid = rows < N
    _gat_pre(i, x0_ref[...], w_ref, atts_ref, attd_ref, hm_ref, wl_ref, bl_ref,
             h_ref, ass_ref, asd_ref, mxs_ref, mxd_ref, skn_ref, valid)


def _tc_gat_first(x0, w, atts, attd, hm, wl, bl):
    ins, outs, oshapes = _pre_specs(64)
    return pl.pallas_call(
        _tc_gat_first_body,
        grid=(NTB,),
        in_specs=[pl.BlockSpec((TB, 64), lambda i: (i, 0))] + ins,
        out_specs=outs,
        out_shape=oshapes,
    )(x0, w, atts, attd, hm, wl, bl)


def _gat_post(i, ra_ref, dn_ref, sk_ref, bp_ref, ex_ref,
              gmx_ref, gsm_ref, valid):
    den_exp = jnp.dot(dn_ref[...], ex_ref[...], preferred_element_type=f32)
    x = ra_ref[...] / (den_exp + 1e-16) + bp_ref[...] + sk_ref[...]

    @pl.when(i == 0)
    def _():
        gmx_ref[...] = jnp.full((1, HID), -1e30, f32)
        gsm_ref[...] = jnp.zeros((1, HID), f32)

    gmx_ref[...] = jnp.maximum(
        gmx_ref[...], jnp.max(jnp.where(valid, x, -1e30), axis=0, keepdims=True))
    gsm_ref[...] = gsm_ref[...] + jnp.sum(
        jnp.where(valid, x, 0.0), axis=0, keepdims=True)
    return x


def _post_specs():
    ins = [
        pl.BlockSpec((TB, HID), lambda i: (i, 0)),
        pl.BlockSpec((TB, 16), lambda i: (i, 0)),
        pl.BlockSpec((TB, HID), lambda i: (i, 0)),
        pl.BlockSpec((1, HID), lambda i: (0, 0)),
        pl.BlockSpec((16, HID), lambda i: (0, 0)),
    ]
    outs = [
        pl.BlockSpec((1, HID), lambda i: (0, 0)),
        pl.BlockSpec((1, HID), lambda i: (0, 0)),
    ]
    oshapes = [
        jax.ShapeDtypeStruct((1, HID), f32),
        jax.ShapeDtypeStruct((1, HID), f32),
    ]
    return ins, outs, oshapes


def _tc_gat_mid_body(ra_ref, dn_ref, sk_ref, bp_ref, ex_ref,
                     w_ref, atts_ref, attd_ref, hm_ref, wl_ref, bl_ref,
                     gmx_ref, gsm_ref, h_ref, ass_ref, asd_ref,
                     mxs_ref, mxd_ref, skn_ref):
    i = pl.program_id(0)
    rows = i * TB + lax.broadcasted_iota(i32, (TB, 1), 0)
    valid = rows < N
    x = _gat_post(i, ra_ref, dn_ref, sk_ref, bp_ref, ex_ref,
                  gmx_ref, gsm_ref, valid)
    xr = jnp.where(valid, jnp.maximum(x, 0.0), 0.0)
    _gat_pre(i, xr, w_ref, atts_ref, attd_ref, hm_ref, wl_ref, bl_ref,
             h_ref, ass_ref, asd_ref, mxs_ref, mxd_ref, skn_ref, valid)


def _tc_gat_mid(ra, dn, sk, bp, ex, w, atts, attd, hm, wl, bl):
    pins, pouts, pshapes = _post_specs()
    ins, outs, oshapes = _pre_specs(HID)
    return pl.pallas_call(
        _tc_gat_mid_body,
        grid=(NTB,),
        in_specs=pins + ins,
        out_specs=pouts + outs,
        out_shape=pshapes + oshapes,
    )(ra, dn, sk, bp, ex, w, atts, attd, hm, wl, bl)


def _tc_gat_last_body(ra_ref, dn_ref, sk_ref, bp_ref, ex_ref,
                      gmx_ref, gsm_ref):
    i = pl.program_id(0)
    rows = i * TB + lax.broadcasted_iota(i32, (TB, 1), 0)
    valid = rows < N
    _gat_post(i, ra_ref, dn_ref, sk_ref, bp_ref, ex_ref,
              gmx_ref, gsm_ref, valid)


def _tc_gat_last(ra, dn, sk, bp, ex):
    pins, pouts, pshapes = _post_specs()
    return pl.pallas_call(
        _tc_gat_last_body,
        grid=(NTB,),
        in_specs=pins,
        out_specs=pouts,
        out_shape=pshapes,
    )(ra, dn, sk, bp, ex)


@functools.partial(
    pl.kernel,
    out_type=(
        jax.ShapeDtypeStruct((NPAD, HID), f32),
        jax.ShapeDtypeStruct((NPAD * 16,), f32),
    ),
    mesh=_mesh,
    compiler_params=pltpu.CompilerParams(needs_layout_passes=False),
    scratch_types=[
        pltpu.VMEM((C4,), i32),
        pltpu.VMEM((C4,), i32),
        pltpu.VMEM((C4 + 16,), i32),
        pltpu.VMEM((C4 + 16,), i32),
        pltpu.VMEM((G4,), i32),
        pltpu.VMEM((G4,), i32),
        pltpu.VMEM((G4, HID), f32),
        pltpu.VMEM((G4, 128), f32),
        pltpu.VMEM((RNG * 16,), f32),
        pltpu.VMEM((RNG * 16,), f32),
        pltpu.VMEM((RNG, HID), f32),
        pltpu.VMEM((16,), f32),
        pltpu.VMEM((16,), f32),
        pltpu.SemaphoreType.DMA,
        pltpu.SemaphoreType.DMA,
    ],
)
def _sc_gat(s2, d2, aspk, as16d, a16, h, raw, den,
            dbuf, sbuf, pdl, psv, gidxs, gidx8, hg, asg, adl, dacc, acc,
            a16v, exb, sem1, sem2):
    lo = _wid() * RNG
    hi = lo + RNG
    zero = jnp.zeros((16,), f32)
    zeroi = jnp.zeros((16,), i32)

    pltpu.sync_copy(a16.at[0], a16v)
    avec = a16v[...]
    pltpu.sync_copy(as16d.at[pl.ds(lo * 16, RNG * 16)], adl)

    def zr(r, _):
        for k in range(16):
            acc[r, pl.ds(16 * k, 16)] = zero
        dacc[pl.ds(r * 16, 16)] = zero
        return 0

    lax.fori_loop(0, RNG, zr, 0)

    def zp(j, _):
        psv[pl.ds(j * 16, 16)] = zeroi
        return 0

    lax.fori_loop(0, (C4 + 16) // 16, zp, 0)

    HV = [lax.iota(i32, 16) * 0 + (k // 2) for k in range(16)]

    def chunk(ci, _):
        b = ci * C4
        pltpu.sync_copy(d2.at[pl.ds(b, C4)], dbuf)
        pltpu.sync_copy(s2.at[pl.ds(b, C4)], sbuf)

        def cvec(j, cnt):
            d = dbuf[pl.ds(j * 16, 16)]
            s = sbuf[pl.ds(j * 16, 16)]
            m = (d >= lo) & (d < hi)
            pos = plsc.cumsum(m.astype(i32))
            plsc.store_scatter(pdl, [cnt + pos - 1], d - lo, mask=m)
            plsc.store_scatter(psv, [cnt + pos - 1], s, mask=m)
            return cnt + pos[15]

        m_total = lax.fori_loop(0, C4 // 16, cvec, 0)

        def gbatch(k, _):
            @pl.when(k * G4 < m_total)
            def _():
                def bidx(t_, _2):
                    v = psv[pl.ds(k * G4 + t_ * 16, 16)]
                    gidxs[pl.ds(t_ * 16, 16)] = v
                    gidx8[pl.ds(t_ * 16, 16)] = v >> 3
                    return 0

                lax.fori_loop(0, G4 // 16, bidx, 0)
                cp1 = pltpu.async_copy(h.at[gidxs], hg, sem1)
                cp2 = pltpu.async_copy(aspk.at[gidx8], asg, sem2)
                cp1.wait()
                cp2.wait()

                def edge(j, _2):
                    idx = k * G4 + j

                    @pl.when(idx < m_total)
                    def _3():
                        sv = psv[pl.ds(idx, 16)][0]
                        dl = pdl[pl.ds(idx, 16)][0]
                        z = (asg[j, pl.ds((sv & 7) * 16, 16)]
                             + adl[pl.ds(dl * 16, 16)])
                        z = jnp.maximum(z, 0.2 * z) - avec
                        ex = jnp.exp(z)
                        dacc[pl.ds(dl * 16, 16)] = dacc[pl.ds(dl * 16, 16)] + ex
                        exb[pl.ds(0, 16)] = ex
                        for k2 in range(16):
                            bc = plsc.load_gather(exb, [HV[k2]])
                            acc[dl, pl.ds(16 * k2, 16)] = (
                                acc[dl, pl.ds(16 * k2, 16)]
                                + hg[j, pl.ds(16 * k2, 16)] * bc)
                    return 0

                lax.fori_loop(0, G4, edge, 0)
            return 0

        lax.fori_loop(0, C4 // G4 + 1, gbatch, 0)
        return 0

    lax.fori_loop(0, NC4, chunk, 0)
    pltpu.sync_copy(acc, raw.at[pl.ds(lo, RNG)])
    pltpu.sync_copy(dacc, den.at[pl.ds(lo * 16, RNG * 16)])


def _tc_final_body(gm_ref, gs_ref, w1_ref, b1_ref, w2_ref, b2_ref, w3_ref,
                   b3_ref, o_ref):
    gmp = gm_ref[...].reshape(1, 4 * HID)
    gap = gs_ref[...].reshape(1, 4 * HID) * (1.0 / N)
    g = jnp.concatenate([gmp, gap], axis=1)
    h = jnp.dot(g, w1_ref[...], preferred_element_type=f32) + b1_ref[...]
    h = jnp.dot(h, w2_ref[...], preferred_element_type=f32) + b2_ref[...]
    h = jnp.dot(h, w3_ref[...], preferred_element_type=f32) + b3_ref[...]
    m = jnp.max(h, axis=1, keepdims=True)
    ex = jnp.exp(h - m)
    lse = jnp.log(jnp.sum(ex, axis=1, keepdims=True)) + m
    o_ref[...] = h - lse


def _tc_final(gm, gs, w1, b1, w2, b2, w3p, b3p):
    return pl.pallas_call(
        _tc_final_body,
        grid=(1,),
        in_specs=[
            pl.BlockSpec((4, HID), lambda i: (0, 0)),
            pl.BlockSpec((4, HID), lambda i: (0, 0)),
            pl.BlockSpec((8 * HID, 512), lambda i: (0, 0)),
            pl.BlockSpec((1, 512), lambda i: (0, 0)),
            pl.BlockSpec((512, 256), lambda i: (0, 0)),
            pl.BlockSpec((1, 256), lambda i: (0, 0)),
            pl.BlockSpec((256, 128), lambda i: (0, 0)),
            pl.BlockSpec((1, 128), lambda i: (0, 0)),
        ],
        out_specs=pl.BlockSpec((1, 128), lambda i: (0, 0)),
        out_shape=jax.ShapeDtypeStruct((1, 128), f32),
    )(gm, gs, w1, b1, w2, b2, w3p, b3p)


def _fold_bn(W, b, g, be):
    s = g / jnp.sqrt(1.0 + 1e-5)
    return W * s[None, :], (b * s + be)[None, :]


def kernel(x, edge_index, batch, params):
    src = edge_index[0].astype(i32)
    dst = edge_index[1].astype(i32)

    (W1, b1, g1, be1), (W2, b2, g2, be2), (W3, b3, g3, be3) = params["edge_mlp"]
    W1f, c1 = _fold_bn(W1, b1, g1, be1)
    W2f, c2 = _fold_bn(W2, b2, g2, be2)
    W3f, c3 = _fold_bn(W3, b3, g3, be3)
    wr = W1f[:128] - W1f[128:]
    wq = W1f[128:]

    hm16 = np.zeros((HID, 16), np.float32)
    for cch in range(HID):
        hm16[cch, cch // 32] = 1.0
        hm16[cch, 8 + cch // 32] = 1.0
    hm16 = jnp.asarray(hm16)
    expd = np.zeros((16, HID), np.float32)
    for j in range(8):
        expd[j, 32 * j:32 * (j + 1)] = 1.0
    expd = jnp.asarray(expd)

    xp = jnp.pad(x, ((0, NPAD - N), (0, 0)))
    loop = jnp.arange(N, dtype=i32)
    padv = jnp.full((EPAD - E - N + EXTRA,), N, i32)
    s2 = jnp.concatenate([src, loop, padv])
    d2 = jnp.concatenate([dst, loop, padv])

    t = _tc_prep(xp, wr, wq, c1)
    h1 = _sc_edge(t, src, dst)
    h3 = _tc_mlp(h1, W2f, c2, W3f, c3)
    h3p = h3.reshape(E // 2, 128)
    x0 = _sc_segmax(dst, h3p)

    def gat_flat(p):
        return (p["W"], p["att_src"].reshape(1, HID),
                p["att_dst"].reshape(1, HID), p["bias"].reshape(1, HID))

    def lin_flat(p):
        return p[0], p[1].reshape(1, HID)

    w_g, atts, attd, bias_prev = gat_flat(params["conv1"])
    wl, bl = lin_flat(params["lin1"])
    h, ass, asd, mxs, mxd, skip = _tc_gat_first(
        x0, w_g, atts, attd, hm16, wl, bl)

    gmx, gsm = [], []
    for li in (2, 3, 4):
        mm = mxs + mxd
        a16 = jnp.maximum(mm, 0.2 * mm)
        asp = ass.reshape(NPAD // 8, 128)
        ra, den = _sc_gat(s2, d2, asp, asd.reshape(NPAD * 16), a16, h)
        den = den.reshape(NPAD, 16)
        w_g, atts, attd, bias = gat_flat(params[f"conv{li}"])
        wl, bl = lin_flat(params[f"lin{li}"])
        (gm, gs, h, ass, asd, mxs, mxd, skip) = _tc_gat_mid(
            ra, den, skip, bias_prev, expd,
            w_g, atts, attd, hm16, wl, bl)
        gmx.append(gm)
        gsm.append(gs)
        bias_prev = bias

    mm = mxs + mxd
    a16 = jnp.maximum(mm, 0.2 * mm)
    asp = ass.reshape(NPAD // 8, 128)
    ra, den = _sc_gat(s2, d2, asp, asd.reshape(NPAD * 16), a16, h)
    den = den.reshape(NPAD, 16)
    gm, gs = _tc_gat_last(ra, den, skip, bias_prev, expd)
    gmx.append(gm)
    gsm.append(gs)

    (Wm1, bm1), (Wm2, bm2), (Wm3, bm3) = params["mlp"]
    w3p = jnp.zeros((256, 128), f32).at[:, :NCLS].set(Wm3)
    b3p = jnp.full((1, 128), -1e30, f32).at[0, :NCLS].set(bm3)
    gm_all = jnp.concatenate(gmx, axis=0)
    gs_all = jnp.concatenate(gsm, axis=0)
    out = _tc_final(gm_all, gs_all, Wm1, bm1.reshape(1, 512),
                    Wm2, bm2.reshape(1, 256), w3p, b3p)
    return out[:, :NCLS]


# speedup vs baseline: 1.0688x; 1.0688x over previous
import functools
import jax
import jax.numpy as jnp
import numpy as np
from jax import lax
from jax.experimental import pallas as pl
from jax.experimental.pallas import tpu as pltpu
from jax.experimental.pallas import tpu_sc as plsc

N = 10000
E = 320000
DIN = 128
HID = 256
HEADS = 8
NCLS = 40

NPAD = 10240
EPAD = 330240
EXTRA = 16
NC, NS = 2, 16
NW = NC * NS
EW1 = E // NW
EW2 = EPAD // NW
CH1 = 200
NCH1 = EW1 // CH1
C4 = 3840
NC4 = EPAD // C4
G4 = 48
RNG = NPAD // NW
ZR = 80
C3 = 4000
NC3 = E // C3
G3 = 128
TB = 2048
NTB = NPAD // TB
EB = 2000
NEB = E // EB

_mesh = plsc.VectorSubcoreMesh(
    core_axis_name="c", subcore_axis_name="s", num_cores=NC, num_subcores=NS)

f32 = jnp.float32
i32 = jnp.int32


def _wid():
    return lax.axis_index("s") * NC + lax.axis_index("c")


def _tc_prep_body(x_ref, wr_ref, wq_ref, c1_ref, t_ref):
    x = x_ref[...]
    r = jnp.dot(x, wr_ref[...], preferred_element_type=f32) + c1_ref[...]
    q = jnp.dot(x, wq_ref[...], preferred_element_type=f32)
    t_ref[...] = jnp.concatenate([r, q], axis=1)


def _tc_prep(xp, wr, wq, c1):
    return pl.pallas_call(
        _tc_prep_body,
        grid=(NTB,),
        in_specs=[
            pl.BlockSpec((TB, DIN), lambda i: (i, 0)),
            pl.BlockSpec((DIN, 64), lambda i: (0, 0)),
            pl.BlockSpec((DIN, 64), lambda i: (0, 0)),
            pl.BlockSpec((1, 64), lambda i: (0, 0)),
        ],
        out_specs=pl.BlockSpec((TB, 128), lambda i: (i, 0)),
        out_shape=jax.ShapeDtypeStruct((NPAD, 128), f32),
    )(xp, wr, wq, c1)


@functools.partial(
    pl.kernel,
    out_type=jax.ShapeDtypeStruct((E, 64), f32),
    mesh=_mesh,
    compiler_params=pltpu.CompilerParams(needs_layout_passes=False),
    scratch_types=[
        pltpu.VMEM((CH1,), i32),
        pltpu.VMEM((CH1,), i32),
        pltpu.VMEM((CH1, 128), f32),
        pltpu.VMEM((CH1, 128), f32),
        pltpu.VMEM((CH1, 64), f32),
        pltpu.SemaphoreType.DMA,
        pltpu.SemaphoreType.DMA,
    ],
)
def _sc_edge(t, src, dst, h1, sidx, didx, db, sb, ob, sem1, sem2):
    base = _wid() * EW1

    def chunk(ci, _):
        b = base + ci * CH1
        pltpu.sync_copy(src.at[pl.ds(b, CH1)], sidx)
        pltpu.sync_copy(dst.at[pl.ds(b, CH1)], didx)
        cp1 = pltpu.async_copy(t.at[didx], db, sem1)
        cp2 = pltpu.async_copy(t.at[sidx], sb, sem2)
        cp1.wait()
        cp2.wait()

        def edge(e, _):
            for k in range(4):
                v = db[e, pl.ds(16 * k, 16)] + sb[e, pl.ds(64 + 16 * k, 16)]
                ob[e, pl.ds(16 * k, 16)] = jnp.maximum(v, 0.0)
            return 0

        lax.fori_loop(0, CH1, edge, 0)
        pltpu.sync_copy(ob, h1.at[pl.ds(b, CH1)])
        return 0

    lax.fori_loop(0, NCH1, chunk, 0)


def _tc_mlp_body(h1_ref, w2_ref, c2_ref, w3_ref, c3_ref, o_ref):
    h2 = jnp.maximum(
        jnp.dot(h1_ref[...], w2_ref[...], preferred_element_type=f32)
        + c2_ref[...], 0.0)
    o_ref[...] = jnp.maximum(
        jnp.dot(h2, w3_ref[...], preferred_element_type=f32) + c3_ref[...], 0.0)


def _tc_mlp(h1, w2, c2, w3, c3):
    return pl.pallas_call(
        _tc_mlp_body,
        grid=(NEB,),
        in_specs=[
            pl.BlockSpec((EB, 64), lambda i: (i, 0)),
            pl.BlockSpec((64, 64), lambda i: (0, 0)),
            pl.BlockSpec((1, 64), lambda i: (0, 0)),
            pl.BlockSpec((64, 64), lambda i: (0, 0)),
            pl.BlockSpec((1, 64), lambda i: (0, 0)),
        ],
        out_specs=pl.BlockSpec((EB, 64), lambda i: (i, 0)),
        out_shape=jax.ShapeDtypeStruct((E, 64), f32),
    )(h1, w2, c2, w3, c3)


@functools.partial(
    pl.kernel,
    out_type=jax.ShapeDtypeStruct((NPAD, 64), f32),
    mesh=_mesh,
    compiler_params=pltpu.CompilerParams(needs_layout_passes=False),
    scratch_types=[
        pltpu.VMEM((C3,), i32),
        pltpu.VMEM((C3 + 16,), i32),
        pltpu.VMEM((G3,), i32),
        pltpu.VMEM((G3, 128), f32),
        pltpu.VMEM((RNG, 64), f32),
        pltpu.SemaphoreType.DMA,
    ],
)
def _sc_segmax(dst, h3p, x0, dbuf, pbuf, gidx, rows, acc, sem):
    lo = _wid() * RNG
    hi = lo + RNG
    zero = jnp.zeros((16,), f32)
    zeroi = jnp.zeros((16,), i32)

    def zr(r, _):
        for k in range(4):
            acc[r, pl.ds(16 * k, 16)] = zero
        return 0

    lax.fori_loop(0, RNG, zr, 0)

    def zp(j, _):
        pbuf[pl.ds(j * 16, 16)] = zeroi
        return 0

    lax.fori_loop(0, (C3 + 16) // 16, zp, 0)

    def chunk(ci, _):
        b = ci * C3
        pltpu.sync_copy(dst.at[pl.ds(b, C3)], dbuf)

        def cvec(j, cnt):
            d = dbuf[pl.ds(j * 16, 16)]
            m = (d >= lo) & (d < hi)
            eid = lax.iota(i32, 16) + (b + j * 16)
            packed = ((d - lo) << 20) | eid
            pos = plsc.cumsum(m.astype(i32))
            plsc.store_scatter(pbuf, [cnt + pos - 1], packed, mask=m)
            return cnt + pos[15]

        m_total = lax.fori_loop(0, C3 // 16, cvec, 0)

        def gbatch(k, _):
            @pl.when(k * G3 < m_total)
            def _():
                def ext(t_, _2):
                    p = pbuf[pl.ds(k * G3 + t_ * 16, 16)]
                    gidx[pl.ds(t_ * 16, 16)] = (p & 0xFFFFF) >> 1
                    return 0

                lax.fori_loop(0, G3 // 16, ext, 0)
                pltpu.async_copy(h3p.at[gidx], rows, sem).wait()

                def edge(j, _2):
                    idx = k * G3 + j

                    @pl.when(idx < m_total)
                    def _3():
                        p = pbuf[pl.ds(idx, 16)][0]
                        dl = p >> 20
                        half = (p & 1) * 64
                        for q in range(4):
                            cur = acc[dl, pl.ds(16 * q, 16)]
                            acc[dl, pl.ds(16 * q, 16)] = jnp.maximum(
                                cur, rows[j, pl.ds(half + 16 * q, 16)])
                    return 0

                lax.fori_loop(0, G3, edge, 0)
            return 0

        lax.fori_loop(0, C3 // G3 + 1, gbatch, 0)
        return 0

    lax.fori_loop(0, NC3, chunk, 0)
    pltpu.sync_copy(acc, x0.at[pl.ds(lo, RNG)])


def _gat_pre(i, xr, w_ref, atts_ref, attd_ref, hm_ref, wl_ref, bl_ref,
             h_ref, ass_ref, asd_ref, mxs_ref, mxd_ref, skn_ref,
             valid):
    h = jnp.dot(xr, w_ref[...], preferred_element_type=f32)
    h_ref[...] = h
    asv = jnp.dot(h * atts_ref[...], hm_ref[...], preferred_element_type=f32)
    adv = jnp.dot(h * attd_ref[...], hm_ref[...], preferred_element_type=f32)
    asv = jnp.where(valid, asv, -1e30)
    adv = jnp.where(valid, adv, -1e30)
    ass_ref[...] = asv
    asd_ref[...] = adv

    @pl.when(i == 0)
    def _():
        mxs_ref[...] = jnp.full((1, 16), -1e30, f32)
        mxd_ref[...] = jnp.full((1, 16), -1e30, f32)

    mxs_ref[...] = jnp.maximum(mxs_ref[...], jnp.max(asv, axis=0, keepdims=True))
    mxd_ref[...] = jnp.maximum(mxd_ref[...], jnp.max(adv, axis=0, keepdims=True))
    skn_ref[...] = jnp.dot(xr, wl_ref[...], preferred_element_type=f32) + bl_ref[...]


def _pre_specs(din):
    ins = [
        pl.BlockSpec((din, HID), lambda i: (0, 0)),
        pl.BlockSpec((1, HID), lambda i: (0, 0)),
        pl.BlockSpec((1, HID), lambda i: (0, 0)),
        pl.BlockSpec((HID, 16), lambda i: (0, 0)),
        pl.BlockSpec((din, HID), lambda i: (0, 0)),
        pl.BlockSpec((1, HID), lambda i: (0, 0)),
    ]
    outs = [
        pl.BlockSpec((TB, HID), lambda i: (i, 0)),
        pl.BlockSpec((TB, 16), lambda i: (i, 0)),
        pl.BlockSpec((TB, 16), lambda i: (i, 0)),
        pl.BlockSpec((1, 16), lambda i: (0, 0)),
        pl.BlockSpec((1, 16), lambda i: (0, 0)),
        pl.BlockSpec((TB, HID), lambda i: (i, 0)),
    ]
    oshapes = [
        jax.ShapeDtypeStruct((NPAD, HID), f32),
        jax.ShapeDtypeStruct((NPAD, 16), f32),
        jax.ShapeDtypeStruct((NPAD, 16), f32),
        jax.ShapeDtypeStruct((1, 16), f32),
        jax.ShapeDtypeStruct((1, 16), f32),
        jax.ShapeDtypeStruct((NPAD, HID), f32),
    ]
    return ins, outs, oshapes


def _tc_gat_first_body(x0_ref, w_ref, atts_ref, attd_ref, hm_ref, wl_ref,
                       bl_ref, h_ref, ass_ref, asd_ref, mxs_ref,
                       mxd_ref, skn_ref):
    i = pl.program_id(0)
    rows = i * TB + lax.broadcasted_iota(i32, (TB, 1), 0)
    valid = rows < N
    _gat_pre(i, x0_ref[...], w_ref, atts_ref, attd_ref, hm_ref, wl_ref, bl_ref,
             h_ref, ass_ref, asd_ref, mxs_ref, mxd_ref, skn_ref, valid)


def _tc_gat_first(x0, w, atts, attd, hm, wl, bl):
    ins, outs, oshapes = _pre_specs(64)
    return pl.pallas_call(
        _tc_gat_first_body,
        grid=(NTB,),
        in_specs=[pl.BlockSpec((TB, 64), lambda i: (i, 0))] + ins,
        out_specs=outs,
        out_shape=oshapes,
    )(x0, w, atts, attd, hm, wl, bl)


def _gat_post(i, ra_ref, dn_ref, sk_ref, bp_ref, ex_ref,
              gmx_ref, gsm_ref, valid):
    den_exp = jnp.dot(dn_ref[...], ex_ref[...], preferred_element_type=f32)
    x = ra_ref[...] / (den_exp + 1e-16) + bp_ref[...] + sk_ref[...]

    @pl.when(i == 0)
    def _():
        gmx_ref[...] = jnp.full((1, HID), -1e30, f32)
        gsm_ref[...] = jnp.zeros((1, HID), f32)

    gmx_ref[...] = jnp.maximum(
        gmx_ref[...], jnp.max(jnp.where(valid, x, -1e30), axis=0, keepdims=True))
    gsm_ref[...] = gsm_ref[...] + jnp.sum(
        jnp.where(valid, x, 0.0), axis=0, keepdims=True)
    return x


def _post_specs():
    ins = [
        pl.BlockSpec((TB, HID), lambda i: (i, 0)),
        pl.BlockSpec((TB, 16), lambda i: (i, 0)),
        pl.BlockSpec((TB, HID), lambda i: (i, 0)),
        pl.BlockSpec((1, HID), lambda i: (0, 0)),
        pl.BlockSpec((16, HID), lambda i: (0, 0)),
    ]
    outs = [
        pl.BlockSpec((1, HID), lambda i: (0, 0)),
        pl.BlockSpec((1, HID), lambda i: (0, 0)),
    ]
    oshapes = [
        jax.ShapeDtypeStruct((1, HID), f32),
        jax.ShapeDtypeStruct((1, HID), f32),
    ]
    return ins, outs, oshapes


def _tc_gat_mid_body(ra_ref, dn_ref, sk_ref, bp_ref, ex_ref,
                     w_ref, atts_ref, attd_ref, hm_ref, wl_ref, bl_ref,
                     gmx_ref, gsm_ref, h_ref, ass_ref, asd_ref,
                     mxs_ref, mxd_ref, skn_ref):
    i = pl.program_id(0)
    rows = i * TB + lax.broadcasted_iota(i32, (TB, 1), 0)
    valid = rows < N
    x = _gat_post(i, ra_ref, dn_ref, sk_ref, bp_ref, ex_ref,
                  gmx_ref, gsm_ref, valid)
    xr = jnp.where(valid, jnp.maximum(x, 0.0), 0.0)
    _gat_pre(i, xr, w_ref, atts_ref, attd_ref, hm_ref, wl_ref, bl_ref,
             h_ref, ass_ref, asd_ref, mxs_ref, mxd_ref, skn_ref, valid)


def _tc_gat_mid(ra, dn, sk, bp, ex, w, atts, attd, hm, wl, bl):
    pins, pouts, pshapes = _post_specs()
    ins, outs, oshapes = _pre_specs(HID)
    return pl.pallas_call(
        _tc_gat_mid_body,
        grid=(NTB,),
        in_specs=pins + ins,
        out_specs=pouts + outs,
        out_shape=pshapes + oshapes,
    )(ra, dn, sk, bp, ex, w, atts, attd, hm, wl, bl)


def _tc_gat_last_body(ra_ref, dn_ref, sk_ref, bp_ref, ex_ref,
                      gmx_ref, gsm_ref):
    i = pl.program_id(0)
    rows = i * TB + lax.broadcasted_iota(i32, (TB, 1), 0)
    valid = rows < N
    _gat_post(i, ra_ref, dn_ref, sk_ref, bp_ref, ex_ref,
              gmx_ref, gsm_ref, valid)


def _tc_gat_last(ra, dn, sk, bp, ex):
    pins, pouts, pshapes = _post_specs()
    return pl.pallas_call(
        _tc_gat_last_body,
        grid=(NTB,),
        in_specs=pins,
        out_specs=pouts,
        out_shape=pshapes,
    )(ra, dn, sk, bp, ex)


@functools.partial(
    pl.kernel,
    out_type=(
        jax.ShapeDtypeStruct((NPAD, HID), f32),
        jax.ShapeDtypeStruct((NPAD * 16,), f32),
    ),
    mesh=_mesh,
    compiler_params=pltpu.CompilerParams(needs_layout_passes=False),
    scratch_types=[
        pltpu.VMEM((C4,), i32),
        pltpu.VMEM((C4,), i32),
        pltpu.VMEM((C4 + 16,), i32),
        pltpu.VMEM((C4 + 16,), i32),
        pltpu.VMEM((G4,), i32),
        pltpu.VMEM((G4,), i32),
        pltpu.VMEM((G4, HID), f32),
        pltpu.VMEM((G4, 128), f32),
        pltpu.VMEM((RNG * 16,), f32),
        pltpu.VMEM((RNG * 16,), f32),
        pltpu.VMEM((RNG, HID), f32),
        pltpu.VMEM((16,), f32),
        pltpu.VMEM((16,), f32),
        pltpu.SemaphoreType.DMA,
        pltpu.SemaphoreType.DMA,
    ],
)
def _sc_gat(s2, d2, aspk, as16d, a16, h, raw, den,
            dbuf, sbuf, pdl, psv, gidxs, gidx8, hg, asg, adl, dacc, acc,
            a16v, exb, sem1, sem2):
    lo = _wid() * RNG
    hi = lo + RNG
    zero = jnp.zeros((16,), f32)
    zeroi = jnp.zeros((16,), i32)

    pltpu.sync_copy(a16.at[0], a16v)
    avec = a16v[...]
    pltpu.sync_copy(as16d.at[pl.ds(lo * 16, RNG * 16)], adl)

    def zr(r, _):
        for k in range(16):
            acc[r, pl.ds(16 * k, 16)] = zero
        dacc[pl.ds(r * 16, 16)] = zero
        return 0

    lax.fori_loop(0, RNG, zr, 0)

    def zp(j, _):
        psv[pl.ds(j * 16, 16)] = zeroi
        return 0

    lax.fori_loop(0, (C4 + 16) // 16, zp, 0)

    HV = [lax.iota(i32, 16) * 0 + (k // 2) for k in range(16)]

    def chunk(ci, _):
        b = ci * C4
        pltpu.sync_copy(d2.at[pl.ds(b, C4)], dbuf)
        pltpu.sync_copy(s2.at[pl.ds(b, C4)], sbuf)

        def cvec(j, cnt):
            d = dbuf[pl.ds(j * 16, 16)]
            s = sbuf[pl.ds(j * 16, 16)]
            m = (d >= lo) & (d < hi)
            pos = plsc.cumsum(m.astype(i32))
            plsc.store_scatter(pdl, [cnt + pos - 1], d - lo, mask=m)
            plsc.store_scatter(psv, [cnt + pos - 1], s, mask=m)
            return cnt + pos[15]

        m_total = lax.fori_loop(0, C4 // 16, cvec, 0)

        def gbatch(k, _):
            @pl.when(k * G4 < m_total)
            def _():
                def bidx(t_, _2):
                    v = psv[pl.ds(k * G4 + t_ * 16, 16)]
                    gidxs[pl.ds(t_ * 16, 16)] = v
                    gidx8[pl.ds(t_ * 16, 16)] = v >> 3
                    return 0

                lax.fori_loop(0, G4 // 16, bidx, 0)
                cp1 = pltpu.async_copy(h.at[gidxs], hg, sem1)
                cp2 = pltpu.async_copy(aspk.at[gidx8], asg, sem2)
                cp1.wait()
                cp2.wait()

                def edge(j, _2):
                    idx = k * G4 + j

                    @pl.when(idx < m_total)
                    def _3():
                        sv = psv[pl.ds(idx, 16)][0]
                        dl = pdl[pl.ds(idx, 16)][0]
                        z = (asg[j, pl.ds((sv & 7) * 16, 16)]
                             + adl[pl.ds(dl * 16, 16)])
                        z = jnp.maximum(z, 0.2 * z) - avec
                        ex = jnp.exp(z)
                        dacc[pl.ds(dl * 16, 16)] = dacc[pl.ds(dl * 16, 16)] + ex
                        exb[pl.ds(0, 16)] = ex
                        for k2 in range(16):
                            bc = plsc.load_gather(exb, [HV[k2]])
                            acc[dl, pl.ds(16 * k2, 16)] = (
                                acc[dl, pl.ds(16 * k2, 16)]
                                + hg[j, pl.ds(16 * k2, 16)] * bc)
                    return 0

                lax.fori_loop(0, G4, edge, 0)
            return 0

        lax.fori_loop(0, C4 // G4 + 1, gbatch, 0)
        return 0

    lax.fori_loop(0, NC4, chunk, 0)
    pltpu.sync_copy(acc, raw.at[pl.ds(lo, RNG)])
    pltpu.sync_copy(dacc, den.at[pl.ds(lo * 16, RNG * 16)])


def _tc_final_body(gm_ref, gs_ref, w1_ref, b1_ref, w2_ref, b2_ref, w3_ref,
                   b3_ref, o_ref):
    gmp = gm_ref[...].reshape(1, 4 * HID)
    gap = gs_ref[...].reshape(1, 4 * HID) * (1.0 / N)
    g = jnp.concatenate([gmp, gap], axis=1)
    h = jnp.dot(g, w1_ref[...], preferred_element_type=f32) + b1_ref[...]
    h = jnp.dot(h, w2_ref[...], preferred_element_type=f32) + b2_ref[...]
    h = jnp.dot(h, w3_ref[...], preferred_element_type=f32) + b3_ref[...]
    m = jnp.max(h, axis=1, keepdims=True)
    ex = jnp.exp(h - m)
    lse = jnp.log(jnp.sum(ex, axis=1, keepdims=True)) + m
    o_ref[...] = h - lse


def _tc_final(gm, gs, w1, b1, w2, b2, w3p, b3p):
    return pl.pallas_call(
        _tc_final_body,
        grid=(1,),
        in_specs=[
            pl.BlockSpec((4, HID), lambda i: (0, 0)),
            pl.BlockSpec((4, HID), lambda i: (0, 0)),
            pl.BlockSpec((8 * HID, 512), lambda i: (0, 0)),
            pl.BlockSpec((1, 512), lambda i: (0, 0)),
            pl.BlockSpec((512, 256), lambda i: (0, 0)),
            pl.BlockSpec((1, 256), lambda i: (0, 0)),
            pl.BlockSpec((256, 128), lambda i: (0, 0)),
            pl.BlockSpec((1, 128), lambda i: (0, 0)),
        ],
        out_specs=pl.BlockSpec((1, 128), lambda i: (0, 0)),
        out_shape=jax.ShapeDtypeStruct((1, 128), f32),
    )(gm, gs, w1, b1, w2, b2, w3p, b3p)


def _fold_bn(W, b, g, be):
    s = g / jnp.sqrt(1.0 + 1e-5)
    return W * s[None, :], (b * s + be)[None, :]


def kernel(x, edge_index, batch, params):
    src = edge_index[0].astype(i32)
    dst = edge_index[1].astype(i32)

    (W1, b1, g1, be1), (W2, b2, g2, be2), (W3, b3, g3, be3) = params["edge_mlp"]
    W1f, c1 = _fold_bn(W1, b1, g1, be1)
    W2f, c2 = _fold_bn(W2, b2, g2, be2)
    W3f, c3 = _fold_bn(W3, b3, g3, be3)
    wr = W1f[:128] - W1f[128:]
    wq = W1f[128:]

    hm16 = np.zeros((HID, 16), np.float32)
    for cch in range(HID):
        hm16[cch, cch // 32] = 1.0
        hm16[cch, 8 + cch // 32] = 1.0
    hm16 = jnp.asarray(hm16)
    expd = np.zeros((16, HID), np.float32)
    for j in range(8):
        expd[j, 32 * j:32 * (j + 1)] = 1.0
    expd = jnp.asarray(expd)

    xp = jnp.pad(x, ((0, NPAD - N), (0, 0)))
    loop = jnp.arange(N, dtype=i32)
    padv = jnp.full((EPAD - E - N + EXTRA,), N, i32)
    s2 = jnp.concatenate([src, loop, padv])
    d2 = jnp.concatenate([dst, loop, padv])

    t = _tc_prep(xp, wr, wq, c1)
    h1 = _sc_edge(t, src, dst)
    h3 = _tc_mlp(h1, W2f, c2, W3f, c3)
    h3p = h3.reshape(E // 2, 128)
    x0 = _sc_segmax(dst, h3p)

    def gat_flat(p):
        return (p["W"], p["att_src"].reshape(1, HID),
                p["att_dst"].reshape(1, HID), p["bias"].reshape(1, HID))

    def lin_flat(p):
        return p[0], p[1].reshape(1, HID)

    w_g, atts, attd, bias_prev = gat_flat(params["conv1"])
    wl, bl = lin_flat(params["lin1"])
    h, ass, asd, mxs, mxd, skip = _tc_gat_first(
        x0, w_g, atts, attd, hm16, wl, bl)

    gmx, gsm = [], []
    for li in (2, 3, 4):
        mm = mxs + mxd
        a16 = jnp.maximum(mm, 0.2 * mm)
        asp = ass.reshape(NPAD // 8, 128)
        ra, den = _sc_gat(s2, d2, asp, asd.reshape(NPAD * 16), a16, h)
        den = den.reshape(NPAD, 16)
        w_g, atts, attd, bias = gat_flat(params[f"conv{li}"])
        wl, bl = lin_flat(params[f"lin{li}"])
        (gm, gs, h, ass, asd, mxs, mxd, skip) = _tc_gat_mid(
            ra, den, skip, bias_prev, expd,
            w_g, atts, attd, hm16, wl, bl)
        gmx.append(gm)
        gsm.append(gs)
        bias_prev = bias

    mm = mxs + mxd
    a16 = jnp.maximum(mm, 0.2 * mm)
    asp = ass.reshape(NPAD // 8, 128)
    ra, den = _sc_gat(s2, d2, asp, asd.reshape(NPAD * 16), a16, h)
    den = den.reshape(NPAD, 16)
    gm, gs = _tc_gat_last(ra, den, skip, bias_prev, expd)
    gmx.append(gm)
    gsm.append(gs)

    (Wm1, bm1), (Wm2, bm2), (Wm3, bm3) = params["mlp"]
    w3p = jnp.zeros((256, 128), f32).at[:, :NCLS].set(Wm3)
    b3p = jnp.full((1, 128), -1e30, f32).at[0, :NCLS].set(bm3)
    gm_all = jnp.concatenate(gmx, axis=0)
    gs_all = jnp.concatenate(gsm, axis=0)
    out = _tc_final(gm_all, gs_all, Wm1, bm1.reshape(1, 512),
                    Wm2, bm2.reshape(1, 256), w3p, b3p)
    return out[:, :NCLS]
